# baseline (device time: 93286 ns/iter reference)
import jax
import jax.numpy as jnp
from jax import lax
from jax.experimental import pallas as pl
from jax.experimental.pallas import tpu as pltpu

T = 512
TH = T // 2
D = 1024
V_LOC = 8192
KC = 8
CW = V_LOC // KC


def kernel(x, W):
    def body(x_ref, w_hbm, out_ref, x_bf, w_buf,
             my_stats, xn_stats, yn_stats, diag_stats,
             w_sems, sx_sems, sy_sems, fx_sems, fy_sems,
             rx_sems, ry_sems, rd_sems, st_send, st_recv):
        my_x = lax.axis_index("x")
        my_y = lax.axis_index("y")
        xn = (1 - my_x, my_y)
        yn = (my_x, 1 - my_y)

        my_rows = pl.ds(my_x * TH, TH)
        other_rows = pl.ds((1 - my_x) * TH, TH)
        my_col0 = my_y * V_LOC
        other_col0 = (1 - my_y) * V_LOC

        def rcopy(src, dst, ssem, rsem, dev):
            return pltpu.make_async_remote_copy(
                src_ref=src, dst_ref=dst, send_sem=ssem, recv_sem=rsem,
                device_id=dev, device_id_type=pl.DeviceIdType.MESH,
            )

        barrier_sem = pltpu.get_barrier_semaphore()
        for nbr in (xn, yn):
            pl.semaphore_signal(
                barrier_sem, inc=1, device_id=nbr,
                device_id_type=pl.DeviceIdType.MESH,
            )

        x_bf[...] = x_ref[my_rows, :].astype(jnp.bfloat16)

        def w_dma(k):
            return pltpu.make_async_copy(
                w_hbm.at[:, pl.ds(k * CW, CW)],
                w_buf.at[k % 2],
                w_sems.at[k % 2],
            )

        w_dma(0).start()
        sxs, sys_, fwds = [], [], [None] * KC
        for k in range(KC):
            if k + 1 < KC:
                w_dma(k + 1).start()
            w_dma(k).wait()
            wk = w_buf[k % 2].astype(jnp.bfloat16)
            logits = jnp.dot(
                x_bf[...], wk, preferred_element_type=jnp.float32
            )
            m = logits.max(axis=1, keepdims=True)
            e = jnp.exp(logits - m)
            my_stats[0, :, k : k + 1] = m
            my_stats[1, :, k : k + 1] = e.sum(axis=1, keepdims=True)
            col = pl.ds(my_col0 + k * CW, CW)
            out_ref[my_rows, col] = e.astype(jnp.bfloat16)
            if k == 0:
                pl.semaphore_wait(barrier_sem, 2)
            blk = out_ref.at[my_rows, col]
            sx = rcopy(blk, blk, sx_sems.at[k], rx_sems.at[k], xn)
            sy = rcopy(blk, blk, sy_sems.at[k], ry_sems.at[k], yn)
            sx.start()
            sy.start()
            sxs.append(sx)
            sys_.append(sy)

        st_x = rcopy(my_stats, xn_stats, st_send.at[0], st_recv.at[0], xn)
        st_y = rcopy(my_stats, yn_stats, st_send.at[1], st_recv.at[1], yn)
        st_x.start()
        st_y.start()

        xrecvs = []
        yrecvs = []
        for k in range(KC):
            xcol = pl.ds(my_col0 + k * CW, CW)
            xblk = out_ref.at[other_rows, xcol]
            xr = rcopy(xblk, xblk, sx_sems.at[k], rx_sems.at[k], xn)
            ycol = pl.ds(other_col0 + k * CW, CW)
            yblk = out_ref.at[my_rows, ycol]
            yr = rcopy(yblk, yblk, sy_sems.at[k], ry_sems.at[k], yn)
            xrecvs.append(xr)
            yrecvs.append(yr)
            if k % 2 == 1:
                xr.wait_recv()
                fwd = rcopy(xblk, xblk, fy_sems.at[k], rd_sems.at[k], yn)
            else:
                yr.wait_recv()
                fwd = rcopy(yblk, yblk, fx_sems.at[k], rd_sems.at[k], xn)
            fwd.start()
            fwds[k] = fwd

        rcopy(my_stats, xn_stats, st_send.at[0], st_recv.at[0], xn).wait_recv()
        st_f = rcopy(xn_stats, diag_stats, st_send.at[2], st_recv.at[2], yn)
        st_f.start()

        rcopy(my_stats, yn_stats, st_send.at[1], st_recv.at[1], yn).wait_recv()
        rcopy(my_stats, diag_stats, st_send.at[2], st_recv.at[2], yn).wait_recv()

        def factors(a_stats, b_stats):
            am, asum = a_stats[0], a_stats[1]
            bm, bsum = b_stats[0], b_stats[1]
            m_fin = jnp.maximum(
                am.max(axis=1, keepdims=True), bm.max(axis=1, keepdims=True)
            )
            ea = jnp.exp(am - m_fin)
            eb = jnp.exp(bm - m_fin)
            s_fin = (asum * ea).sum(axis=1, keepdims=True) + (
                bsum * eb
            ).sum(axis=1, keepdims=True)
            inv = 1.0 / s_fin
            return ea * inv, eb * inv

        fac_my, fac_yn = factors(my_stats, yn_stats)
        fac_xn, fac_diag = factors(xn_stats, diag_stats)

        def rescale(rows, col, fac, k):
            out_ref[rows, col] = (
                out_ref[rows, col].astype(jnp.float32) * fac[:, k : k + 1]
            ).astype(jnp.bfloat16)

        for k in range(KC):
            sxs[k].wait_send()
            sys_[k].wait_send()
            rescale(my_rows, pl.ds(my_col0 + k * CW, CW), fac_my, k)

        for k in range(KC):
            if k % 2 == 0:
                xrecvs[k].wait_recv()
            else:
                fwds[k].wait_send()
            rescale(other_rows, pl.ds(my_col0 + k * CW, CW), fac_xn, k)
            if k % 2 == 1:
                yrecvs[k].wait_recv()
            else:
                fwds[k].wait_send()
            rescale(my_rows, pl.ds(other_col0 + k * CW, CW), fac_yn, k)

        for k in range(KC):
            dcol = pl.ds(other_col0 + k * CW, CW)
            dblk = out_ref.at[other_rows, dcol]
            rcopy(dblk, dblk, fx_sems.at[k], rd_sems.at[k], xn).wait_recv()
            rescale(other_rows, dcol, fac_diag, k)

        st_x.wait_send()
        st_y.wait_send()
        st_f.wait_send()

    stat_shape = pltpu.VMEM((2, TH, KC), jnp.float32)
    return pl.pallas_call(
        body,
        out_shape=jax.ShapeDtypeStruct((T, 2 * V_LOC), jnp.bfloat16),
        in_specs=[
            pl.BlockSpec(memory_space=pltpu.VMEM),
            pl.BlockSpec(memory_space=pl.ANY),
        ],
        out_specs=pl.BlockSpec(memory_space=pltpu.VMEM),
        scratch_shapes=[
            pltpu.VMEM((TH, D), jnp.bfloat16),
            pltpu.VMEM((2, D, CW), jnp.float32),
            stat_shape,
            stat_shape,
            stat_shape,
            stat_shape,
            pltpu.SemaphoreType.DMA((2,)),
            pltpu.SemaphoreType.DMA((KC,)),
            pltpu.SemaphoreType.DMA((KC,)),
            pltpu.SemaphoreType.DMA((KC,)),
            pltpu.SemaphoreType.DMA((KC,)),
            pltpu.SemaphoreType.DMA((KC,)),
            pltpu.SemaphoreType.DMA((KC,)),
            pltpu.SemaphoreType.DMA((KC,)),
            pltpu.SemaphoreType.DMA((3,)),
            pltpu.SemaphoreType.DMA((3,)),
        ],
        compiler_params=pltpu.CompilerParams(collective_id=0),
    )(x, W)


# device time: 78012 ns/iter; 1.1958x vs baseline; 1.1958x over previous
import jax
import jax.numpy as jnp
from jax import lax
from jax.experimental import pallas as pl
from jax.experimental.pallas import tpu as pltpu

T = 512
D = 1024
V_LOC = 8192
KS = 16
SW = V_LOC // KS
KC = 8
CW = V_LOC // KC


def kernel(x, W):
    def body(x_ref, w_hbm, out_ref, x_bf, w_buf, my_stats, peer_stats,
             w_sems, sy_sems, f_sems, ry_sems, fr_sems, st_sems):
        my_x = lax.axis_index("x")
        my_y = lax.axis_index("y")
        xn = (1 - my_x, my_y)
        yn = (my_x, 1 - my_y)

        my_col0 = my_y * V_LOC
        other_col0 = (1 - my_y) * V_LOC

        def rcopy(src, dst, ssem, rsem, dev):
            return pltpu.make_async_remote_copy(
                src_ref=src, dst_ref=dst, send_sem=ssem, recv_sem=rsem,
                device_id=dev, device_id_type=pl.DeviceIdType.MESH,
            )

        barrier_sem = pltpu.get_barrier_semaphore()
        for nbr in (xn, yn):
            pl.semaphore_signal(
                barrier_sem, inc=1, device_id=nbr,
                device_id_type=pl.DeviceIdType.MESH,
            )

        x_bf[...] = x_ref[...].astype(jnp.bfloat16)

        def w_dma(j):
            return pltpu.make_async_copy(
                w_hbm.at[:, pl.ds(j * SW, SW)],
                w_buf.at[j % 2],
                w_sems.at[j % 2],
            )

        w_dma(0).start()
        sys_ = [None] * KC
        for j in range(KS):
            if j + 1 < KS:
                w_dma(j + 1).start()
            w_dma(j).wait()
            wj = w_buf[j % 2].astype(jnp.bfloat16)
            logits = jnp.dot(
                x_bf[...], wj, preferred_element_type=jnp.float32
            )
            m = logits.max(axis=1, keepdims=True)
            e = jnp.exp(logits - m)
            my_stats[0, :, j : j + 1] = m
            my_stats[1, :, j : j + 1] = e.sum(axis=1, keepdims=True)
            out_ref[:, pl.ds(my_col0 + j * SW, SW)] = e.astype(jnp.bfloat16)
            if j == 0:
                pl.semaphore_wait(barrier_sem, 2)
            if j % 2 == 1:
                k = j // 2
                blk = out_ref.at[:, pl.ds(my_col0 + k * CW, CW)]
                send = rcopy(blk, blk, sy_sems.at[k], ry_sems.at[k], yn)
                @pl.when(k % 2 == my_x)
                def _():
                    send.start()
                sys_[k] = send

        st = rcopy(my_stats, peer_stats, st_sems.at[0], st_sems.at[1], yn)
        st.start()

        fwds = [None] * KC
        recvs = [None] * KC
        for k in range(KC):
            blk = out_ref.at[:, pl.ds(other_col0 + k * CW, CW)]
            direct = rcopy(blk, blk, sy_sems.at[k], ry_sems.at[k], yn)
            fwd = rcopy(blk, blk, f_sems.at[k], fr_sems.at[k], xn)
            recvs[k] = rcopy(blk, blk, f_sems.at[k], fr_sems.at[k], xn)

            @pl.when(k % 2 == my_x)
            def _():
                direct.wait_recv()
                fwd.start()

            fwds[k] = fwd

        rcopy(my_stats, peer_stats, st_sems.at[0], st_sems.at[1], yn).wait_recv()

        mm, ms = my_stats[0], my_stats[1]
        pm, ps = peer_stats[0], peer_stats[1]
        m_fin = jnp.maximum(
            mm.max(axis=1, keepdims=True), pm.max(axis=1, keepdims=True)
        )
        em = jnp.exp(mm - m_fin)
        ep = jnp.exp(pm - m_fin)
        s_fin = (ms * em).sum(axis=1, keepdims=True) + (ep * ps).sum(
            axis=1, keepdims=True
        )
        inv = 1.0 / s_fin
        fac_mine = em * inv
        fac_peer = ep * inv

        def rescale(col0, fac, j):
            sl = pl.ds(col0 + j * SW, SW)
            out_ref[:, sl] = (
                out_ref[:, sl].astype(jnp.float32) * fac[:, j : j + 1]
            ).astype(jnp.bfloat16)

        for k in range(KC):
            send = sys_[k]

            @pl.when(k % 2 == my_x)
            def _():
                send.wait_send()

            for j in (2 * k, 2 * k + 1):
                rescale(my_col0, fac_mine, j)

        for k in range(KC):
            @pl.when(k % 2 == my_x)
            def _():
                fwds[k].wait_send()

            @pl.when(k % 2 != my_x)
            def _():
                recvs[k].wait_recv()

            for j in (2 * k, 2 * k + 1):
                rescale(other_col0, fac_peer, j)

        st.wait_send()

    stat_shape = pltpu.VMEM((2, T, KS), jnp.float32)
    return pl.pallas_call(
        body,
        out_shape=jax.ShapeDtypeStruct((T, 2 * V_LOC), jnp.bfloat16),
        in_specs=[
            pl.BlockSpec(memory_space=pltpu.VMEM),
            pl.BlockSpec(memory_space=pl.ANY),
        ],
        out_specs=pl.BlockSpec(memory_space=pltpu.VMEM),
        scratch_shapes=[
            pltpu.VMEM((T, D), jnp.bfloat16),
            pltpu.VMEM((2, D, SW), jnp.float32),
            stat_shape,
            stat_shape,
            pltpu.SemaphoreType.DMA((2,)),
            pltpu.SemaphoreType.DMA((KC,)),
            pltpu.SemaphoreType.DMA((KC,)),
            pltpu.SemaphoreType.DMA((KC,)),
            pltpu.SemaphoreType.DMA((KC,)),
            pltpu.SemaphoreType.DMA((2,)),
        ],
        compiler_params=pltpu.CompilerParams(collective_id=0),
    )(x, W)
